# baseline (speedup 1.0000x reference)
import jax
import jax.numpy as jnp
from jax import lax
from jax.experimental import pallas as pl
from jax.experimental.pallas import tpu as pltpu
from jax.experimental.pallas import tpu_sc as plsc

NUM_EMB = 1000
EMB_DIM = 128
BATCH = 16384
LANES = 16
VREGS_PER_ROW = EMB_DIM // LANES
EPS = 1e-6
MARGIN = 1.0

_info = plsc.get_sparse_core_info()
_NC, _NS = _info.num_cores, _info.num_subcores
NW = _NC * _NS
B_PER_W = BATCH // NW
CHUNK = 64
N_CHUNKS = B_PER_W // CHUNK
OUT_ROWS = BATCH // 8
OUT_ROWS_W = B_PER_W // 8


def _sc_body(table_hbm, a_hbm, p_hbm, n_hbm, sq_ap_hbm, sq_an_hbm,
             idx_a, idx_p, idx_n,
             ra0, rp0, rn0, ra1, rp1, rn1,
             sq_ap_v, sq_an_v, sem0, sem1):
    wid = lax.axis_index("s") * _NC + lax.axis_index("c")
    base = wid * B_PER_W

    pltpu.sync_copy(a_hbm.at[pl.ds(base, B_PER_W)], idx_a)
    pltpu.sync_copy(p_hbm.at[pl.ds(base, B_PER_W)], idx_p)
    pltpu.sync_copy(n_hbm.at[pl.ds(base, B_PER_W)], idx_n)

    bufs = ((ra0, rp0, rn0, sem0), (ra1, rp1, rn1, sem1))

    def issue(c):
        ba, bp, bn, sem = bufs[c & 1]
        s = pl.ds(c * CHUNK, CHUNK)
        return (pltpu.async_copy(table_hbm.at[idx_a.at[s]], ba, sem),
                pltpu.async_copy(table_hbm.at[idx_p.at[s]], bp, sem),
                pltpu.async_copy(table_hbm.at[idx_n.at[s]], bn, sem))

    inflight = issue(0)
    for c in range(N_CHUNKS):
        ba, bp, bn, _ = bufs[c & 1]
        nxt = issue(c + 1) if c + 1 < N_CHUNKS else None
        for d in inflight:
            d.wait()
        inflight = nxt

        @plsc.parallel_loop(0, CHUNK, step=1, unroll=4)
        def row_body(i, c=c, ba=ba, bp=bp, bn=bn):
            acc_ap = jnp.zeros((LANES,), jnp.float32)
            acc_an = jnp.zeros((LANES,), jnp.float32)
            for v in range(VREGS_PER_ROW):
                a = ba[i, pl.ds(v * LANES, LANES)] + EPS
                d_ap = a - bp[i, pl.ds(v * LANES, LANES)]
                d_an = a - bn[i, pl.ds(v * LANES, LANES)]
                acc_ap = acc_ap + d_ap * d_ap
                acc_an = acc_an + d_an * d_an
            j = c * CHUNK + i
            row = j >> 3
            lane = (j & 7) * LANES
            sq_ap_v[row, pl.ds(lane, LANES)] = acc_ap
            sq_an_v[row, pl.ds(lane, LANES)] = acc_an

    pltpu.sync_copy(sq_ap_v, sq_ap_hbm.at[pl.ds(wid * OUT_ROWS_W, OUT_ROWS_W)])
    pltpu.sync_copy(sq_an_v, sq_an_hbm.at[pl.ds(wid * OUT_ROWS_W, OUT_ROWS_W)])


_sc_gather_dist = pl.kernel(
    _sc_body,
    mesh=plsc.VectorSubcoreMesh(core_axis_name="c", subcore_axis_name="s"),
    compiler_params=pltpu.CompilerParams(use_tc_tiling_on_sc=False),
    out_type=[
        jax.ShapeDtypeStruct((OUT_ROWS, EMB_DIM), jnp.float32),
        jax.ShapeDtypeStruct((OUT_ROWS, EMB_DIM), jnp.float32),
    ],
    scratch_types=[
        pltpu.VMEM((B_PER_W,), jnp.int32),
        pltpu.VMEM((B_PER_W,), jnp.int32),
        pltpu.VMEM((B_PER_W,), jnp.int32),
        pltpu.VMEM((CHUNK, EMB_DIM), jnp.float32),
        pltpu.VMEM((CHUNK, EMB_DIM), jnp.float32),
        pltpu.VMEM((CHUNK, EMB_DIM), jnp.float32),
        pltpu.VMEM((CHUNK, EMB_DIM), jnp.float32),
        pltpu.VMEM((CHUNK, EMB_DIM), jnp.float32),
        pltpu.VMEM((CHUNK, EMB_DIM), jnp.float32),
        pltpu.VMEM((OUT_ROWS_W, EMB_DIM), jnp.float32),
        pltpu.VMEM((OUT_ROWS_W, EMB_DIM), jnp.float32),
        pltpu.SemaphoreType.DMA,
        pltpu.SemaphoreType.DMA,
    ],
)


def _tc_loss_body(sq_ap_ref, sq_an_ref, out_ref):
    k = lax.broadcasted_iota(jnp.int32, (EMB_DIM, 8), 0) // LANES
    s = lax.broadcasted_iota(jnp.int32, (EMB_DIM, 8), 1)
    seg = (k == s).astype(jnp.float32)
    d2_ap = jnp.dot(sq_ap_ref[...], seg, preferred_element_type=jnp.float32)
    d2_an = jnp.dot(sq_an_ref[...], seg, preferred_element_type=jnp.float32)
    t = jnp.maximum(jnp.sqrt(d2_ap) - jnp.sqrt(d2_an) + MARGIN, 0.0)
    out_ref[0, 0] = jnp.sum(t) / BATCH


def kernel(anchor, positive, negative, table):
    sq_ap, sq_an = _sc_gather_dist(
        table, anchor.astype(jnp.int32), positive.astype(jnp.int32),
        negative.astype(jnp.int32))
    loss = pl.pallas_call(
        _tc_loss_body,
        out_shape=jax.ShapeDtypeStruct((1, 1), jnp.float32),
        in_specs=[
            pl.BlockSpec(memory_space=pltpu.VMEM),
            pl.BlockSpec(memory_space=pltpu.VMEM),
        ],
        out_specs=pl.BlockSpec(memory_space=pltpu.SMEM),
    )(sq_ap, sq_an)
    return loss.reshape(())


# speedup vs baseline: 1.4135x; 1.4135x over previous
import jax
import jax.numpy as jnp
from jax import lax
from jax.experimental import pallas as pl
from jax.experimental.pallas import tpu as pltpu
from jax.experimental.pallas import tpu_sc as plsc

NUM_EMB = 1000
EMB_DIM = 128
BATCH = 16384
LANES = 16
EPS = 1e-6
MARGIN = 1.0

_info = plsc.get_sparse_core_info()
_NC, _NS = _info.num_cores, _info.num_subcores
NW = _NC * _NS
B_PER_W = BATCH // NW
N_VECS = B_PER_W // LANES
N_STREAMS = B_PER_W // 128


def _tc_gram_body(t_ref, d2_ref):
    t = t_ref[...]
    g = lax.dot_general(t, t, (((1,), (1,)), ((), ())),
                        preferred_element_type=jnp.float32)
    n = jnp.sum(t * t, axis=1)
    s = jnp.sum(t, axis=1)
    m = n + (2.0 * EPS) * s + EMB_DIM * EPS * EPS
    w = n - (2.0 * EPS) * s
    d2_ref[...] = m[:, None] + w[None, :] - 2.0 * g


_tc_gram = pl.pallas_call(
    _tc_gram_body,
    out_shape=jax.ShapeDtypeStruct((NUM_EMB, NUM_EMB), jnp.float32),
    in_specs=[pl.BlockSpec(memory_space=pltpu.VMEM)],
    out_specs=pl.BlockSpec(memory_space=pltpu.VMEM),
)


def _rsqrt_newton(x):
    i = lax.bitcast_convert_type(x, jnp.int32)
    r = lax.bitcast_convert_type(jnp.int32(0x5F3759DF) - (i >> 1), jnp.float32)
    hx = 0.5 * x
    for _ in range(3):
        r = r * (1.5 - hx * r * r)
    return r


def _sc_body(d2_hbm, a_hbm, p_hbm, n_hbm, out_hbm,
             idx_a, idx_p, idx_n, fa_ap, fa_an, g_ap, g_an, accv, sem):
    wid = lax.axis_index("s") * _NC + lax.axis_index("c")
    base = wid * B_PER_W

    pltpu.sync_copy(a_hbm.at[pl.ds(base, B_PER_W)], idx_a)
    pltpu.sync_copy(p_hbm.at[pl.ds(base, B_PER_W)], idx_p)
    pltpu.sync_copy(n_hbm.at[pl.ds(base, B_PER_W)], idx_n)

    @plsc.parallel_loop(0, N_VECS, step=1, unroll=2)
    def fa_body(t):
        sl = pl.ds(t * LANES, LANES)
        a1000 = idx_a[sl] * NUM_EMB
        fa_ap[sl] = a1000 + idx_p[sl]
        fa_an[sl] = a1000 + idx_n[sl]

    descs = []
    for t in range(N_STREAMS):
        sl = pl.ds(t * 128, 128)
        descs.append(pltpu.async_copy(d2_hbm.at[fa_ap.at[sl]], g_ap.at[sl], sem))
        descs.append(pltpu.async_copy(d2_hbm.at[fa_an.at[sl]], g_an.at[sl], sem))
    for d in descs:
        d.wait()

    def loss_body(t, acc):
        sl = pl.ds(t * LANES, LANES)
        x_ap = jnp.maximum(g_ap[sl], 1e-12)
        x_an = jnp.maximum(g_an[sl], 1e-12)
        d_ap = x_ap * _rsqrt_newton(x_ap)
        d_an = x_an * _rsqrt_newton(x_an)
        return acc + jnp.maximum(d_ap - d_an + MARGIN, 0.0)

    acc = lax.fori_loop(0, N_VECS, loss_body, jnp.zeros((LANES,), jnp.float32))
    accv[0, :] = acc
    pltpu.sync_copy(accv, out_hbm.at[pl.ds(wid, 1)])


_sc_pair_loss = pl.kernel(
    _sc_body,
    mesh=plsc.VectorSubcoreMesh(core_axis_name="c", subcore_axis_name="s"),
    compiler_params=pltpu.CompilerParams(use_tc_tiling_on_sc=False),
    out_type=jax.ShapeDtypeStruct((NW, LANES), jnp.float32),
    scratch_types=[
        pltpu.VMEM((B_PER_W,), jnp.int32),
        pltpu.VMEM((B_PER_W,), jnp.int32),
        pltpu.VMEM((B_PER_W,), jnp.int32),
        pltpu.VMEM((B_PER_W,), jnp.int32),
        pltpu.VMEM((B_PER_W,), jnp.int32),
        pltpu.VMEM((B_PER_W,), jnp.float32),
        pltpu.VMEM((B_PER_W,), jnp.float32),
        pltpu.VMEM((1, LANES), jnp.float32),
        pltpu.SemaphoreType.DMA,
    ],
)


def kernel(anchor, positive, negative, table):
    d2 = _tc_gram(table)
    d2_flat = d2.reshape(-1)
    parts = _sc_pair_loss(
        d2_flat, anchor.astype(jnp.int32), positive.astype(jnp.int32),
        negative.astype(jnp.int32))
    return jnp.sum(parts) / BATCH


# speedup vs baseline: 1.4486x; 1.0248x over previous
import jax
import jax.numpy as jnp
from jax import lax
from jax.experimental import pallas as pl
from jax.experimental.pallas import tpu as pltpu
from jax.experimental.pallas import tpu_sc as plsc

NUM_EMB = 1000
EMB_DIM = 128
BATCH = 16384
LANES = 16
EPS = 1e-6
MARGIN = 1.0

_info = plsc.get_sparse_core_info()
_NC, _NS = _info.num_cores, _info.num_subcores
NW = _NC * _NS
B_PER_W = BATCH // NW
N_VECS = B_PER_W // LANES
N_STREAMS = B_PER_W // 128


V_PAD = 1024
N_CT = V_PAD // EMB_DIM


def _tc_gram_body(tf_ref, ts_ref, d2_ref):
    tf = tf_ref[...]
    ts = ts_ref[...]
    g = lax.dot_general(tf, ts, (((1,), (1,)), ((), ())),
                        preferred_element_type=jnp.float32)
    n = jnp.sum(tf * tf, axis=1, keepdims=True)
    s = jnp.sum(tf, axis=1, keepdims=True)
    m = n + (2.0 * EPS) * s + EMB_DIM * EPS * EPS
    tst = jnp.transpose(ts)
    w = (jnp.sum(tst * tst, axis=0, keepdims=True)
         - (2.0 * EPS) * jnp.sum(tst, axis=0, keepdims=True))
    d2_ref[0] = m + w - 2.0 * g


_tc_gram = pl.pallas_call(
    _tc_gram_body,
    grid=(N_CT,),
    out_shape=jax.ShapeDtypeStruct((N_CT, V_PAD, EMB_DIM), jnp.float32),
    in_specs=[
        pl.BlockSpec((V_PAD, EMB_DIM), lambda t: (0, 0)),
        pl.BlockSpec((EMB_DIM, EMB_DIM), lambda t: (t, 0)),
    ],
    out_specs=pl.BlockSpec((1, V_PAD, EMB_DIM), lambda t: (t, 0, 0)),
)


def _rsqrt_newton(x):
    i = lax.bitcast_convert_type(x, jnp.int32)
    r = lax.bitcast_convert_type(jnp.int32(0x5F3759DF) - (i >> 1), jnp.float32)
    hx = 0.5 * x
    for _ in range(3):
        r = r * (1.5 - hx * r * r)
    return r


def _sc_body(d2_hbm, a_hbm, p_hbm, n_hbm, out_hbm,
             idx_a, idx_p, idx_n, fa_ap, fa_an, g_ap, g_an, accv, sem):
    wid = lax.axis_index("s") * _NC + lax.axis_index("c")
    base = wid * B_PER_W

    pltpu.sync_copy(a_hbm.at[pl.ds(base, B_PER_W)], idx_a)
    pltpu.sync_copy(p_hbm.at[pl.ds(base, B_PER_W)], idx_p)
    pltpu.sync_copy(n_hbm.at[pl.ds(base, B_PER_W)], idx_n)

    @plsc.parallel_loop(0, N_VECS, step=1, unroll=2)
    def fa_body(t):
        sl = pl.ds(t * LANES, LANES)
        abase = idx_a[sl] << 7
        p = idx_p[sl]
        fa_ap[sl] = ((p >> 7) << 17) + abase + (p & 127)
        n = idx_n[sl]
        fa_an[sl] = ((n >> 7) << 17) + abase + (n & 127)

    descs = []
    for t in range(N_STREAMS):
        sl = pl.ds(t * 128, 128)
        descs.append(pltpu.async_copy(d2_hbm.at[fa_ap.at[sl]], g_ap.at[sl], sem))
        descs.append(pltpu.async_copy(d2_hbm.at[fa_an.at[sl]], g_an.at[sl], sem))
    for d in descs:
        d.wait()

    def loss_body(t, acc):
        sl = pl.ds(t * LANES, LANES)
        x_ap = jnp.maximum(g_ap[sl], 1e-12)
        x_an = jnp.maximum(g_an[sl], 1e-12)
        d_ap = x_ap * _rsqrt_newton(x_ap)
        d_an = x_an * _rsqrt_newton(x_an)
        return acc + jnp.maximum(d_ap - d_an + MARGIN, 0.0)

    acc = lax.fori_loop(0, N_VECS, loss_body, jnp.zeros((LANES,), jnp.float32))
    accv[0, :] = acc
    pltpu.sync_copy(accv, out_hbm.at[pl.ds(wid, 1)])


_sc_pair_loss = pl.kernel(
    _sc_body,
    mesh=plsc.VectorSubcoreMesh(core_axis_name="c", subcore_axis_name="s"),
    compiler_params=pltpu.CompilerParams(use_tc_tiling_on_sc=False),
    out_type=jax.ShapeDtypeStruct((NW, LANES), jnp.float32),
    scratch_types=[
        pltpu.VMEM((B_PER_W,), jnp.int32),
        pltpu.VMEM((B_PER_W,), jnp.int32),
        pltpu.VMEM((B_PER_W,), jnp.int32),
        pltpu.VMEM((B_PER_W,), jnp.int32),
        pltpu.VMEM((B_PER_W,), jnp.int32),
        pltpu.VMEM((B_PER_W,), jnp.float32),
        pltpu.VMEM((B_PER_W,), jnp.float32),
        pltpu.VMEM((1, LANES), jnp.float32),
        pltpu.SemaphoreType.DMA,
    ],
)


def kernel(anchor, positive, negative, table):
    table_pad = jnp.pad(table, ((0, V_PAD - NUM_EMB), (0, 0)))
    d2 = _tc_gram(table_pad, table_pad)
    d2_flat = d2.reshape(-1)
    parts = _sc_pair_loss(
        d2_flat, anchor.astype(jnp.int32), positive.astype(jnp.int32),
        negative.astype(jnp.int32))
    return jnp.sum(parts) / BATCH


# speedup vs baseline: 1.7381x; 1.1998x over previous
import jax
import jax.numpy as jnp
from jax import lax
from jax.experimental import pallas as pl
from jax.experimental.pallas import tpu as pltpu
from jax.experimental.pallas import tpu_sc as plsc

NUM_EMB = 1000
EMB_DIM = 128
BATCH = 16384
LANES = 16
EPS = 1e-6
MARGIN = 1.0

_info = plsc.get_sparse_core_info()
_NC, _NS = _info.num_cores, _info.num_subcores
NW = _NC * _NS
B_PER_W = BATCH // NW
N_VECS = B_PER_W // LANES
N_STREAMS = B_PER_W // 128


V_PAD = 1024
N_CT = V_PAD // EMB_DIM


def _tc_gram_body(tf_ref, d2_ref):
    tf = tf_ref[...]
    n = jnp.sum(tf * tf, axis=1, keepdims=True)
    s = jnp.sum(tf, axis=1, keepdims=True)
    m = n + (2.0 * EPS) * s + EMB_DIM * EPS * EPS
    tfp = jnp.pad(tf, ((0, V_PAD - NUM_EMB), (0, 0)))
    for t in range(N_CT):
        ts = tfp[t * EMB_DIM:(t + 1) * EMB_DIM, :]
        g = lax.dot_general(tf, ts, (((1,), (1,)), ((), ())),
                            preferred_element_type=jnp.float32)
        tst = jnp.transpose(ts)
        w = (jnp.sum(tst * tst, axis=0, keepdims=True)
             - (2.0 * EPS) * jnp.sum(tst, axis=0, keepdims=True))
        d2_ref[t] = m + w - 2.0 * g


_tc_gram = pl.pallas_call(
    _tc_gram_body,
    out_shape=jax.ShapeDtypeStruct((N_CT, NUM_EMB, EMB_DIM), jnp.float32),
    in_specs=[pl.BlockSpec(memory_space=pltpu.VMEM)],
    out_specs=pl.BlockSpec(memory_space=pltpu.VMEM),
)


def _rsqrt_newton(x):
    i = lax.bitcast_convert_type(x, jnp.int32)
    r = lax.bitcast_convert_type(jnp.int32(0x5F3759DF) - (i >> 1), jnp.float32)
    hx = 0.5 * x
    for _ in range(3):
        r = r * (1.5 - hx * r * r)
    return r


def _sc_body(d2_hbm, a_hbm, p_hbm, n_hbm, out_hbm,
             idx_a, idx_p, idx_n, fa_ap, fa_an, g_ap, g_an, accv, sem):
    wid = lax.axis_index("s") * _NC + lax.axis_index("c")
    base = wid * B_PER_W

    pltpu.sync_copy(a_hbm.at[pl.ds(base, B_PER_W)], idx_a)
    pltpu.sync_copy(p_hbm.at[pl.ds(base, B_PER_W)], idx_p)
    pltpu.sync_copy(n_hbm.at[pl.ds(base, B_PER_W)], idx_n)

    @plsc.parallel_loop(0, N_VECS, step=1, unroll=2)
    def fa_body(t):
        sl = pl.ds(t * LANES, LANES)
        abase = idx_a[sl] << 7
        p = idx_p[sl]
        fa_ap[sl] = (p >> 7) * (NUM_EMB * EMB_DIM) + abase + (p & 127)
        n = idx_n[sl]
        fa_an[sl] = (n >> 7) * (NUM_EMB * EMB_DIM) + abase + (n & 127)

    descs = []
    for t in range(N_STREAMS):
        sl = pl.ds(t * 128, 128)
        descs.append(pltpu.async_copy(d2_hbm.at[fa_ap.at[sl]], g_ap.at[sl], sem))
        descs.append(pltpu.async_copy(d2_hbm.at[fa_an.at[sl]], g_an.at[sl], sem))
    for d in descs:
        d.wait()

    def loss_body(t, acc):
        sl = pl.ds(t * LANES, LANES)
        x_ap = jnp.maximum(g_ap[sl], 1e-12)
        x_an = jnp.maximum(g_an[sl], 1e-12)
        d_ap = x_ap * _rsqrt_newton(x_ap)
        d_an = x_an * _rsqrt_newton(x_an)
        return acc + jnp.maximum(d_ap - d_an + MARGIN, 0.0)

    acc = lax.fori_loop(0, N_VECS, loss_body, jnp.zeros((LANES,), jnp.float32))
    accv[0, :] = acc
    pltpu.sync_copy(accv, out_hbm.at[pl.ds(wid, 1)])


_sc_pair_loss = pl.kernel(
    _sc_body,
    mesh=plsc.VectorSubcoreMesh(core_axis_name="c", subcore_axis_name="s"),
    compiler_params=pltpu.CompilerParams(use_tc_tiling_on_sc=False),
    out_type=jax.ShapeDtypeStruct((NW, LANES), jnp.float32),
    scratch_types=[
        pltpu.VMEM((B_PER_W,), jnp.int32),
        pltpu.VMEM((B_PER_W,), jnp.int32),
        pltpu.VMEM((B_PER_W,), jnp.int32),
        pltpu.VMEM((B_PER_W,), jnp.int32),
        pltpu.VMEM((B_PER_W,), jnp.int32),
        pltpu.VMEM((B_PER_W,), jnp.float32),
        pltpu.VMEM((B_PER_W,), jnp.float32),
        pltpu.VMEM((1, LANES), jnp.float32),
        pltpu.SemaphoreType.DMA,
    ],
)


def kernel(anchor, positive, negative, table):
    d2 = _tc_gram(table)
    d2_flat = d2.reshape(-1)
    parts = _sc_pair_loss(
        d2_flat, anchor.astype(jnp.int32), positive.astype(jnp.int32),
        negative.astype(jnp.int32))
    return jnp.sum(parts) / BATCH


# speedup vs baseline: 1.7384x; 1.0002x over previous
import jax
import jax.numpy as jnp
from jax import lax
from jax.experimental import pallas as pl
from jax.experimental.pallas import tpu as pltpu
from jax.experimental.pallas import tpu_sc as plsc

NUM_EMB = 1000
EMB_DIM = 128
BATCH = 16384
LANES = 16
EPS = 1e-6
MARGIN = 1.0

_info = plsc.get_sparse_core_info()
_NC, _NS = _info.num_cores, _info.num_subcores
NW = _NC * _NS
B_PER_W = BATCH // NW
N_VECS = B_PER_W // LANES
N_STREAMS = B_PER_W // 128


V_PAD = 1024
N_CT = V_PAD // EMB_DIM


def _tc_gram_body(tf_ref, d2_ref):
    tf = tf_ref[...]
    n = jnp.sum(tf * tf, axis=1, keepdims=True)
    s = jnp.sum(tf, axis=1, keepdims=True)
    m = n + (2.0 * EPS) * s + EMB_DIM * EPS * EPS
    tfp = jnp.pad(tf, ((0, V_PAD - NUM_EMB), (0, 0)))
    for t in range(N_CT):
        ts = tfp[t * EMB_DIM:(t + 1) * EMB_DIM, :]
        g = lax.dot_general(tf, ts, (((1,), (1,)), ((), ())),
                            preferred_element_type=jnp.float32)
        tst = jnp.transpose(ts)
        w = (jnp.sum(tst * tst, axis=0, keepdims=True)
             - (2.0 * EPS) * jnp.sum(tst, axis=0, keepdims=True))
        d2_ref[t] = m + w - 2.0 * g


_tc_gram = pl.pallas_call(
    _tc_gram_body,
    out_shape=jax.ShapeDtypeStruct((N_CT, NUM_EMB, EMB_DIM), jnp.float32),
    in_specs=[pl.BlockSpec(memory_space=pltpu.VMEM)],
    out_specs=pl.BlockSpec(memory_space=pltpu.VMEM),
)


def _rsqrt_newton(x):
    i = lax.bitcast_convert_type(x, jnp.int32)
    r = lax.bitcast_convert_type(jnp.int32(0x5F3759DF) - (i >> 1), jnp.float32)
    hx = 0.5 * x
    for _ in range(3):
        r = r * (1.5 - hx * r * r)
    return r


def _sc_body(d2_hbm, a_hbm, p_hbm, n_hbm, out_hbm,
             idx_a, idx_p, idx_n, fa_ap, fa_an, g_ap, g_an, accv, sem):
    wid = lax.axis_index("s") * _NC + lax.axis_index("c")
    base = wid * B_PER_W

    pltpu.sync_copy(a_hbm.at[pl.ds(base, B_PER_W)], idx_a)
    pltpu.sync_copy(p_hbm.at[pl.ds(base, B_PER_W)], idx_p)
    pltpu.sync_copy(n_hbm.at[pl.ds(base, B_PER_W)], idx_n)

    @plsc.parallel_loop(0, N_VECS, step=1, unroll=2)
    def fa_body(t):
        sl = pl.ds(t * LANES, LANES)
        abase = idx_a[sl] << 7
        p = idx_p[sl]
        fa_ap[sl] = (p >> 7) * (NUM_EMB * EMB_DIM) + abase + (p & 127)
        n = idx_n[sl]
        fa_an[sl] = (n >> 7) * (NUM_EMB * EMB_DIM) + abase + (n & 127)

    descs = []
    for t in range(N_STREAMS):
        sl = pl.ds(t * 128, 128)
        descs.append(pltpu.async_copy(d2_hbm.at[fa_ap.at[sl]], g_ap.at[sl], sem))
        descs.append(pltpu.async_copy(d2_hbm.at[fa_an.at[sl]], g_an.at[sl], sem))
    for d in descs:
        d.wait()

    def loss_body(t, acc):
        sl = pl.ds(t * LANES, LANES)
        x_ap = jnp.maximum(g_ap[sl], 1e-12)
        x_an = jnp.maximum(g_an[sl], 1e-12)
        d_ap = x_ap * _rsqrt_newton(x_ap)
        d_an = x_an * _rsqrt_newton(x_an)
        return acc + jnp.maximum(d_ap - d_an + MARGIN, 0.0)

    acc = lax.fori_loop(0, N_VECS, loss_body, jnp.zeros((LANES,), jnp.float32))
    accv[...] = acc
    pltpu.sync_copy(accv, out_hbm.at[pl.ds(wid * LANES, LANES)])


_sc_pair_loss = pl.kernel(
    _sc_body,
    mesh=plsc.VectorSubcoreMesh(core_axis_name="c", subcore_axis_name="s"),
    compiler_params=pltpu.CompilerParams(use_tc_tiling_on_sc=False),
    out_type=jax.ShapeDtypeStruct((NW * LANES,), jnp.float32),
    scratch_types=[
        pltpu.VMEM((B_PER_W,), jnp.int32),
        pltpu.VMEM((B_PER_W,), jnp.int32),
        pltpu.VMEM((B_PER_W,), jnp.int32),
        pltpu.VMEM((B_PER_W,), jnp.int32),
        pltpu.VMEM((B_PER_W,), jnp.int32),
        pltpu.VMEM((B_PER_W,), jnp.float32),
        pltpu.VMEM((B_PER_W,), jnp.float32),
        pltpu.VMEM((LANES,), jnp.float32),
        pltpu.SemaphoreType.DMA,
    ],
)


def kernel(anchor, positive, negative, table):
    d2 = _tc_gram(table)
    d2_flat = d2.reshape(-1)
    parts = _sc_pair_loss(
        d2_flat, anchor.astype(jnp.int32), positive.astype(jnp.int32),
        negative.astype(jnp.int32))
    return jnp.sum(parts) / BATCH


# speedup vs baseline: 1.8033x; 1.0373x over previous
import jax
import jax.numpy as jnp
from jax import lax
from jax.experimental import pallas as pl
from jax.experimental.pallas import tpu as pltpu
from jax.experimental.pallas import tpu_sc as plsc

NUM_EMB = 1000
EMB_DIM = 128
BATCH = 16384
LANES = 16
EPS = 1e-6
MARGIN = 1.0

_info = plsc.get_sparse_core_info()
_NC, _NS = _info.num_cores, _info.num_subcores
NW = _NC * _NS
B_PER_W = BATCH // NW
N_VECS = B_PER_W // LANES
N_STREAMS = B_PER_W // 128


V_PAD = 1024
N_CT = V_PAD // EMB_DIM


def _tc_gram_body(tf_ref, d2_ref):
    tf = tf_ref[...]
    n = jnp.sum(tf * tf, axis=1, keepdims=True)
    s = jnp.sum(tf, axis=1, keepdims=True)
    m = n + (2.0 * EPS) * s + EMB_DIM * EPS * EPS
    tfp = jnp.pad(tf, ((0, V_PAD - NUM_EMB), (0, 0)))
    for t in range(N_CT):
        ts = tfp[t * EMB_DIM:(t + 1) * EMB_DIM, :]
        g = lax.dot_general(tf, ts, (((1,), (1,)), ((), ())),
                            preferred_element_type=jnp.float32)
        tst = jnp.transpose(ts)
        w = (jnp.sum(tst * tst, axis=0, keepdims=True)
             - (2.0 * EPS) * jnp.sum(tst, axis=0, keepdims=True))
        d2_ref[t] = m + w - 2.0 * g


_tc_gram = pl.pallas_call(
    _tc_gram_body,
    out_shape=jax.ShapeDtypeStruct((N_CT, NUM_EMB, EMB_DIM), jnp.float32),
    in_specs=[pl.BlockSpec(memory_space=pltpu.VMEM)],
    out_specs=pl.BlockSpec(memory_space=pltpu.VMEM),
)


def _rsqrt_newton(x):
    i = lax.bitcast_convert_type(x, jnp.int32)
    r = lax.bitcast_convert_type(jnp.int32(0x5F3759DF) - (i >> 1), jnp.float32)
    hx = 0.5 * x
    for _ in range(3):
        r = r * (1.5 - hx * r * r)
    return r


def _sc_body(d2_hbm, a_hbm, p_hbm, n_hbm, out_hbm,
             idx_a, idx_p, idx_n, fa_ap, fa_an, g_ap, g_an, accv, sem):
    wid = lax.axis_index("s") * _NC + lax.axis_index("c")
    base = wid * B_PER_W

    ca = pltpu.async_copy(a_hbm.at[pl.ds(base, B_PER_W)], idx_a, sem)
    cp = pltpu.async_copy(p_hbm.at[pl.ds(base, B_PER_W)], idx_p, sem)
    cn = pltpu.async_copy(n_hbm.at[pl.ds(base, B_PER_W)], idx_n, sem)
    ca.wait()
    cp.wait()
    cn.wait()

    @plsc.parallel_loop(0, N_VECS, step=1, unroll=2)
    def fa_body(t):
        sl = pl.ds(t * LANES, LANES)
        abase = idx_a[sl] << 7
        p = idx_p[sl]
        fa_ap[sl] = (p >> 7) * (NUM_EMB * EMB_DIM) + abase + (p & 127)
        n = idx_n[sl]
        fa_an[sl] = (n >> 7) * (NUM_EMB * EMB_DIM) + abase + (n & 127)

    descs = []
    for t in range(N_STREAMS):
        sl = pl.ds(t * 128, 128)
        descs.append(pltpu.async_copy(d2_hbm.at[fa_ap.at[sl]], g_ap.at[sl], sem))
        descs.append(pltpu.async_copy(d2_hbm.at[fa_an.at[sl]], g_an.at[sl], sem))
    for d in descs:
        d.wait()

    def loss_body(t, acc):
        sl = pl.ds(t * LANES, LANES)
        x_ap = jnp.maximum(g_ap[sl], 1e-12)
        x_an = jnp.maximum(g_an[sl], 1e-12)
        d_ap = x_ap * _rsqrt_newton(x_ap)
        d_an = x_an * _rsqrt_newton(x_an)
        return acc + jnp.maximum(d_ap - d_an + MARGIN, 0.0)

    acc = lax.fori_loop(0, N_VECS, loss_body, jnp.zeros((LANES,), jnp.float32))
    accv[...] = acc
    pltpu.sync_copy(accv, out_hbm.at[pl.ds(wid * LANES, LANES)])


_sc_pair_loss = pl.kernel(
    _sc_body,
    mesh=plsc.VectorSubcoreMesh(core_axis_name="c", subcore_axis_name="s"),
    compiler_params=pltpu.CompilerParams(use_tc_tiling_on_sc=False),
    out_type=jax.ShapeDtypeStruct((NW * LANES,), jnp.float32),
    scratch_types=[
        pltpu.VMEM((B_PER_W,), jnp.int32),
        pltpu.VMEM((B_PER_W,), jnp.int32),
        pltpu.VMEM((B_PER_W,), jnp.int32),
        pltpu.VMEM((B_PER_W,), jnp.int32),
        pltpu.VMEM((B_PER_W,), jnp.int32),
        pltpu.VMEM((B_PER_W,), jnp.float32),
        pltpu.VMEM((B_PER_W,), jnp.float32),
        pltpu.VMEM((LANES,), jnp.float32),
        pltpu.SemaphoreType.DMA,
    ],
)


def kernel(anchor, positive, negative, table):
    d2 = _tc_gram(table)
    d2_flat = d2.reshape(-1)
    parts = _sc_pair_loss(
        d2_flat, anchor.astype(jnp.int32), positive.astype(jnp.int32),
        negative.astype(jnp.int32))
    return jnp.sum(parts) / BATCH


# speedup vs baseline: 1.8126x; 1.0052x over previous
import jax
import jax.numpy as jnp
from jax import lax
from jax.experimental import pallas as pl
from jax.experimental.pallas import tpu as pltpu
from jax.experimental.pallas import tpu_sc as plsc

NUM_EMB = 1000
EMB_DIM = 128
BATCH = 16384
LANES = 16
EPS = 1e-6
MARGIN = 1.0

_info = plsc.get_sparse_core_info()
_NC, _NS = _info.num_cores, _info.num_subcores
NW = _NC * _NS
B_PER_W = BATCH // NW
N_VECS = B_PER_W // LANES
N_STREAMS = B_PER_W // 128


V_PAD = 1024
N_CT = V_PAD // EMB_DIM


def _tc_gram_body(tf_ref, d2_ref):
    tf = tf_ref[...]
    n = jnp.sum(tf * tf, axis=1, keepdims=True)
    s = jnp.sum(tf, axis=1, keepdims=True)
    m = n + (2.0 * EPS) * s + EMB_DIM * EPS * EPS
    tfp = jnp.pad(tf, ((0, V_PAD - NUM_EMB), (0, 0)))
    g = lax.dot_general(tf, tfp, (((1,), (1,)), ((), ())),
                        preferred_element_type=jnp.float32)
    tt = jnp.transpose(tfp)
    w = (jnp.sum(tt * tt, axis=0, keepdims=True)
         - (2.0 * EPS) * jnp.sum(tt, axis=0, keepdims=True))
    for t in range(N_CT):
        lo, hi = t * EMB_DIM, (t + 1) * EMB_DIM
        d2_ref[t] = m + w[:, lo:hi] - 2.0 * g[:, lo:hi]


_tc_gram = pl.pallas_call(
    _tc_gram_body,
    out_shape=jax.ShapeDtypeStruct((N_CT, NUM_EMB, EMB_DIM), jnp.float32),
    in_specs=[pl.BlockSpec(memory_space=pltpu.VMEM)],
    out_specs=pl.BlockSpec(memory_space=pltpu.VMEM),
)


def _rsqrt_newton(x):
    i = lax.bitcast_convert_type(x, jnp.int32)
    r = lax.bitcast_convert_type(jnp.int32(0x5F3759DF) - (i >> 1), jnp.float32)
    hx = 0.5 * x
    for _ in range(3):
        r = r * (1.5 - hx * r * r)
    return r


def _sc_body(d2_hbm, a_hbm, p_hbm, n_hbm, out_hbm,
             idx_a, idx_p, idx_n, fa_ap, fa_an, g_ap, g_an, accv, sem):
    wid = lax.axis_index("s") * _NC + lax.axis_index("c")
    base = wid * B_PER_W

    ca = pltpu.async_copy(a_hbm.at[pl.ds(base, B_PER_W)], idx_a, sem)
    cp = pltpu.async_copy(p_hbm.at[pl.ds(base, B_PER_W)], idx_p, sem)
    cn = pltpu.async_copy(n_hbm.at[pl.ds(base, B_PER_W)], idx_n, sem)
    ca.wait()
    cp.wait()
    cn.wait()

    @plsc.parallel_loop(0, N_VECS, step=1, unroll=2)
    def fa_body(t):
        sl = pl.ds(t * LANES, LANES)
        abase = idx_a[sl] << 7
        p = idx_p[sl]
        fa_ap[sl] = (p >> 7) * (NUM_EMB * EMB_DIM) + abase + (p & 127)
        n = idx_n[sl]
        fa_an[sl] = (n >> 7) * (NUM_EMB * EMB_DIM) + abase + (n & 127)

    descs = []
    for t in range(N_STREAMS):
        sl = pl.ds(t * 128, 128)
        descs.append(pltpu.async_copy(d2_hbm.at[fa_ap.at[sl]], g_ap.at[sl], sem))
        descs.append(pltpu.async_copy(d2_hbm.at[fa_an.at[sl]], g_an.at[sl], sem))
    for d in descs:
        d.wait()

    def loss_body(t, acc):
        sl = pl.ds(t * LANES, LANES)
        x_ap = jnp.maximum(g_ap[sl], 1e-12)
        x_an = jnp.maximum(g_an[sl], 1e-12)
        d_ap = x_ap * _rsqrt_newton(x_ap)
        d_an = x_an * _rsqrt_newton(x_an)
        return acc + jnp.maximum(d_ap - d_an + MARGIN, 0.0)

    acc = lax.fori_loop(0, N_VECS, loss_body, jnp.zeros((LANES,), jnp.float32))
    accv[...] = acc
    pltpu.sync_copy(accv, out_hbm.at[pl.ds(wid * LANES, LANES)])


_sc_pair_loss = pl.kernel(
    _sc_body,
    mesh=plsc.VectorSubcoreMesh(core_axis_name="c", subcore_axis_name="s"),
    compiler_params=pltpu.CompilerParams(use_tc_tiling_on_sc=False),
    out_type=jax.ShapeDtypeStruct((NW * LANES,), jnp.float32),
    scratch_types=[
        pltpu.VMEM((B_PER_W,), jnp.int32),
        pltpu.VMEM((B_PER_W,), jnp.int32),
        pltpu.VMEM((B_PER_W,), jnp.int32),
        pltpu.VMEM((B_PER_W,), jnp.int32),
        pltpu.VMEM((B_PER_W,), jnp.int32),
        pltpu.VMEM((B_PER_W,), jnp.float32),
        pltpu.VMEM((B_PER_W,), jnp.float32),
        pltpu.VMEM((LANES,), jnp.float32),
        pltpu.SemaphoreType.DMA,
    ],
)


def kernel(anchor, positive, negative, table):
    d2 = _tc_gram(table)
    d2_flat = d2.reshape(-1)
    parts = _sc_pair_loss(
        d2_flat, anchor.astype(jnp.int32), positive.astype(jnp.int32),
        negative.astype(jnp.int32))
    return jnp.sum(parts) / BATCH
